# Initial kernel scaffold; baseline (speedup 1.0000x reference)
import jax
import jax.numpy as jnp
from jax.experimental import pallas as pl


def kernel(x, z, edge_index, W_em, b_em, W_self, W_nbr, W_out, gamma):
    raise NotImplementedError("write your pallas kernel here")


# speedup vs baseline: 3.5163x; 3.5163x over previous
import functools

import jax
import jax.numpy as jnp
from jax import lax
from jax.experimental import pallas as pl
from jax.experimental.pallas import tpu as pltpu, tpu_sc as plsc

N = 50000
E = 1600000
IN_DIM = 128
EM = 32
ODIM = 16
ORB = 4
M = 2 * ORB
D = M * EM
DO = M * ODIM
INV_NBR = 1.0 / 32.0

NP = 53248
NB = 4096
NBLK = 13
BLK_PER_SC = 7
SUB = 256
ACCR = 264
NC = 2
NS = 16
L = 16
EPT = E // NS
CHUNK = 2000
NCHUNKS = EPT // CHUNK
STG = 2048
CAP = EPT + 2 * STG + 96
PCH = 1024
K = 64
ZROWS = 64


ROWS = 512
GRID = NP // ROWS


def _embed_body(x_ref, z_ref, w_ref, b_ref, h_ref, zz_ref):
    w = w_ref[...]
    b = b_ref[...]
    h_ref[...] = jax.nn.sigmoid(
        jnp.dot(x_ref[...], w, preferred_element_type=jnp.float32) + b)
    zz_ref[...] = jax.nn.sigmoid(
        jnp.dot(z_ref[...], w, preferred_element_type=jnp.float32) + b)


def _tc_embed(xp, zp, wcat, bcat):
    return pl.pallas_call(
        _embed_body,
        grid=(GRID,),
        in_specs=[
            pl.BlockSpec((ROWS, IN_DIM), lambda i: (i, 0)),
            pl.BlockSpec((ROWS, IN_DIM), lambda i: (i, 0)),
            pl.BlockSpec((IN_DIM, D), lambda i: (0, 0)),
            pl.BlockSpec((1, D), lambda i: (0, 0)),
        ],
        out_specs=[
            pl.BlockSpec((ROWS, D), lambda i: (i, 0)),
            pl.BlockSpec((ROWS, D), lambda i: (i, 0)),
        ],
        out_shape=[
            jax.ShapeDtypeStruct((NP, D), jnp.float32),
            jax.ShapeDtypeStruct((NP, D), jnp.float32),
        ],
    )(xp, zp, wcat, bcat)


def _update_body(h_ref, a_ref, zz_ref, ws_ref, wn_ref, out_ref):
    agg = a_ref[...] * INV_NBR
    pre = (jnp.dot(h_ref[...], ws_ref[...], preferred_element_type=jnp.float32)
           + jnp.dot(agg, wn_ref[...], preferred_element_type=jnp.float32))
    out_ref[...] = jax.nn.silu(pre) * zz_ref[...]


def _tc_update(h, agg, zz, ws_bd, wn_bd):
    return pl.pallas_call(
        _update_body,
        grid=(GRID,),
        in_specs=[
            pl.BlockSpec((ROWS, D), lambda i: (i, 0)),
            pl.BlockSpec((ROWS, D), lambda i: (i, 0)),
            pl.BlockSpec((ROWS, D), lambda i: (i, 0)),
            pl.BlockSpec((D, D), lambda i: (0, 0)),
            pl.BlockSpec((D, D), lambda i: (0, 0)),
        ],
        out_specs=pl.BlockSpec((ROWS, D), lambda i: (i, 0)),
        out_shape=jax.ShapeDtypeStruct((NP, D), jnp.float32),
    )(h, agg, zz, ws_bd, wn_bd)


def _head_body(h_ref, wo_ref, s_ref, g_ref, out_ref):
    o = jnp.dot(h_ref[...], wo_ref[...], preferred_element_type=jnp.float32)
    t = jnp.abs(jax.nn.silu(o))
    ss = jnp.dot(t * t, s_ref[...], preferred_element_type=jnp.float32)
    out_ref[...] = -(g_ref[...] * t) / (jnp.sqrt(ss) + 1e-8)


def _tc_head(h, wo_bd, smat, gvec):
    return pl.pallas_call(
        _head_body,
        grid=(GRID,),
        in_specs=[
            pl.BlockSpec((ROWS, D), lambda i: (i, 0)),
            pl.BlockSpec((D, DO), lambda i: (0, 0)),
            pl.BlockSpec((DO, DO), lambda i: (0, 0)),
            pl.BlockSpec((1, DO), lambda i: (0, 0)),
        ],
        out_specs=pl.BlockSpec((ROWS, DO), lambda i: (i, 0)),
        out_shape=jax.ShapeDtypeStruct((NP, DO), jnp.float32),
    )(h, wo_bd, smat, gvec)


def _sc_body(h_hbm, src_hbm, dst_hbm,
             agg_hbm, pairs_hbm, counts_hbm,
             src_v, dst_v, stg_v, my_v, pch_v, gsrc_v, gloc_v,
             rows_v, cnt_v, counts_v, acc_v, sem):
    c = lax.axis_index("c")
    s = lax.axis_index("s")
    w = c * NS + s
    lanes = lax.iota(jnp.int32, L)
    zvec = jnp.zeros((L,), jnp.float32)

    def _zero_acc():
        def _zrow(r, _):
            def _zcol(q, _):
                acc_v[r, pl.ds(q * L, L)] = zvec
                return 0
            return lax.fori_loop(0, D // L, _zcol, 0)
        lax.fori_loop(0, ACCR, _zrow, 0)
    _zero_acc()

    def block_body(bi, _):
        b = jnp.where(c == 0, bi, BLK_PER_SC + bi)
        valid_blk = b < NBLK
        base = b * NB

        @pl.when(valid_blk)
        def _():
            def chunk_body(k, total):
                eb = s * EPT + k * CHUNK
                pltpu.sync_copy(src_hbm.at[pl.ds(eb, CHUNK)], src_v)
                pltpu.sync_copy(dst_hbm.at[pl.ds(eb, CHUNK)], dst_v)

                def vreg_body(i, tot):
                    dvec = dst_v[pl.ds(i * L, L)]
                    svec = src_v[pl.ds(i * L, L)]
                    dloc = dvec - base
                    mask = (dloc >= 0) & (dloc < NB)
                    pref = plsc.cumsum(mask.astype(jnp.int32))
                    pos = (tot & (STG - 1)) + pref - 1
                    pack = (svec << 13) | dloc
                    plsc.store_scatter(stg_v, [pos], pack, mask=mask)
                    cnt = jnp.sum(mask.astype(jnp.int32))
                    new = tot + cnt

                    @pl.when((new >> 11) > (tot >> 11))
                    def _():
                        pltpu.sync_copy(
                            stg_v.at[pl.ds(0, STG)],
                            pairs_hbm.at[pl.ds(w * CAP + (tot >> 11) * STG, STG)])
                        stg_v[pl.ds(0, L)] = stg_v[pl.ds(STG, L)]
                    return new
                return lax.fori_loop(0, CHUNK // L, vreg_body, total)
            total = lax.fori_loop(0, NCHUNKS, chunk_body, 0)
            pltpu.sync_copy(stg_v.at[pl.ds(0, STG)],
                            pairs_hbm.at[pl.ds(w * CAP + (total >> 11) * STG, STG)])
            cnt_v[pl.ds(0, L)] = jnp.where(lanes == 0, total, 0)
            pltpu.sync_copy(cnt_v, counts_hbm.at[pl.ds(w * L, L)])
        plsc.subcore_barrier()

        @pl.when(valid_blk)
        def _():
            pltpu.sync_copy(counts_hbm.at[pl.ds(c * NS * L, NS * L)], counts_v)

            def consume(nbat, _):
                def one(j, _):
                    for t in range(K // L):
                        p = my_v[pl.ds(j * K + t * L, L)]
                        gsrc_v[pl.ds(t * L, L)] = p >> 13
                        gloc_v[pl.ds(t * L, L)] = p & 511
                    pltpu.async_copy(h_hbm.at[gsrc_v], rows_v, sem).wait()

                    def edge(i, _):
                        lvec = gloc_v[pl.ds((i >> 4) * L, L)]
                        dl = jnp.sum(jnp.where(lanes == (i & 15), lvec, 0))
                        for q in range(D // L):
                            plsc.addupdate(acc_v.at[dl, pl.ds(q * L, L)],
                                           rows_v[i, pl.ds(q * L, L)])
                        return 0
                    lax.fori_loop(0, K, edge, 0)
                    return 0
                lax.fori_loop(0, nbat, one, 0)
                return 0

            def seg_body(sg, mycnt):
                crow = counts_v[pl.ds(sg * L, L)]
                cnt_s = jnp.sum(jnp.where(lanes == 0, crow, 0))
                nch = (cnt_s + PCH - 1) >> 10

                def pch_body(ch, myc):
                    pltpu.sync_copy(
                        pairs_hbm.at[pl.ds((c * NS + sg) * CAP + ch * PCH,
                                           PCH)],
                        pch_v)
                    left = cnt_s - ch * PCH

                    def fil(i, mc):
                        p = pch_v[pl.ds(i * L, L)]
                        dloc = p & 8191
                        mine = ((dloc >> 8) == s) & (lanes < left - i * L)
                        pref = plsc.cumsum(mine.astype(jnp.int32))
                        pos = mc + pref - 1
                        plsc.store_scatter(my_v, [pos],
                                           (p & ~jnp.int32(8191)) | (dloc & 255),
                                           mask=mine)
                        return mc + jnp.sum(mine.astype(jnp.int32))
                    myc = lax.fori_loop(0, PCH // L, fil, myc)
                    nbat = myc >> 6
                    consume(nbat, 0)
                    rem = myc & 63

                    for t in range(K // L):
                        my_v[pl.ds(t * L, L)] = my_v[pl.ds(nbat * K + t * L, L)]
                    return rem
                return lax.fori_loop(0, nch, pch_body, mycnt)
            mycnt = lax.fori_loop(0, NS, seg_body, 0)

            @pl.when(mycnt > 0)
            def _():
                mix = s * 131 + bi * 29
                for t in range(K // L):
                    dummy = (((lanes + t * L + mix) & 16383) << 13) | \
                        (SUB + ((lanes + t) & 7))
                    pvec = my_v[pl.ds(t * L, L)]
                    keep = (t * L + lanes) < mycnt
                    my_v[pl.ds(t * L, L)] = jnp.where(keep, pvec, dummy)
                consume(1, 0)

            pltpu.sync_copy(acc_v.at[pl.ds(0, SUB)],
                            agg_hbm.at[pl.ds(base + s * SUB, SUB)])
            _zero_acc()
        plsc.subcore_barrier()
        return 0
    lax.fori_loop(0, BLK_PER_SC, block_body, 0)


def _sc_segment_sum(h, src, dst):
    mesh = plsc.VectorSubcoreMesh(core_axis_name="c", subcore_axis_name="s",
                                  num_cores=NC, num_subcores=NS)
    fn = pl.kernel(
        _sc_body,
        out_type=[
            jax.ShapeDtypeStruct((NP, D), jnp.float32),
            jax.ShapeDtypeStruct((NC * NS * CAP,), jnp.int32),
            jax.ShapeDtypeStruct((NC * NS * L,), jnp.int32),
        ],
        mesh=mesh,
        compiler_params=pltpu.CompilerParams(needs_layout_passes=False),
        scratch_types=[
            pltpu.VMEM((CHUNK,), jnp.int32),
            pltpu.VMEM((CHUNK,), jnp.int32),
            pltpu.VMEM((STG + L,), jnp.int32),
            pltpu.VMEM((PCH + K + L,), jnp.int32),
            pltpu.VMEM((PCH,), jnp.int32),
            pltpu.VMEM((K,), jnp.int32),
            pltpu.VMEM((K,), jnp.int32),
            pltpu.VMEM((K, D), jnp.float32),
            pltpu.VMEM((L,), jnp.int32),
            pltpu.VMEM((NS * L,), jnp.int32),
            pltpu.VMEM((ACCR, D), jnp.float32),
            pltpu.SemaphoreType.DMA,
        ],
    )
    return fn(h, src, dst)[0]


def kernel(x, z, edge_index, W_em, b_em, W_self, W_nbr, W_out, gamma):
    src = edge_index[0]
    dst = edge_index[1]

    eye = jnp.eye(M, dtype=jnp.float32)
    wcat = jnp.transpose(W_em, (1, 0, 2)).reshape(IN_DIM, D)
    bcat = b_em.reshape(1, D)
    ws_bd = [jnp.einsum("mij,mn->minj", W_self[:, l], eye).reshape(D, D)
             for l in range(2)]
    wn_bd = [jnp.einsum("mij,mn->minj", W_nbr[:, l], eye).reshape(D, D)
             for l in range(2)]
    wo_bd = jnp.einsum("mij,mn->minj", W_out, eye).reshape(D, DO)
    smat = jnp.einsum("mn,ij->minj", eye,
                      jnp.ones((ODIM, ODIM), jnp.float32)).reshape(DO, DO)
    gvec = jnp.repeat(gamma, ODIM).reshape(1, DO)

    pad = ((0, NP - N), (0, 0))
    xp = jnp.pad(x, pad)
    zp = jnp.pad(z, pad)

    h, zz = _tc_embed(xp, zp, wcat, bcat)
    for l in range(2):
        agg = _sc_segment_sum(h, src, dst)
        h = _tc_update(h, agg, zz, ws_bd[l], wn_bd[l])
    r = _tc_head(h, wo_bd, smat, gvec)

    r = r[:N].reshape(N, M, ODIM).transpose(0, 2, 1)
    r = r.reshape(N, ODIM, 2, ORB).transpose(0, 1, 3, 2)
    return r


# speedup vs baseline: 3.6622x; 1.0415x over previous
import functools

import jax
import jax.numpy as jnp
from jax import lax
from jax.experimental import pallas as pl
from jax.experimental.pallas import tpu as pltpu, tpu_sc as plsc

N = 50000
E = 1600000
IN_DIM = 128
EM = 32
ODIM = 16
ORB = 4
M = 2 * ORB
D = M * EM
DO = M * ODIM
INV_NBR = 1.0 / 32.0

NP = 53248
NB = 4096
NBLK = 13
BLK_PER_SC = 7
SUB = 256
ACCR = 264
NC = 2
NS = 16
L = 16
EPT = E // NS
CHUNK = 4000
NCHUNKS = EPT // CHUNK
STG = 2048
CAP = EPT + 2 * STG + 96
PCH = 2048
K = 64
ZROWS = 64


ROWS = 512
GRID = NP // ROWS


def _embed_body(x_ref, z_ref, w_ref, b_ref, h_ref, zz_ref):
    w = w_ref[...]
    b = b_ref[...]
    h_ref[...] = jax.nn.sigmoid(
        jnp.dot(x_ref[...], w, preferred_element_type=jnp.float32) + b)
    zz_ref[...] = jax.nn.sigmoid(
        jnp.dot(z_ref[...], w, preferred_element_type=jnp.float32) + b)


def _tc_embed(xp, zp, wcat, bcat):
    return pl.pallas_call(
        _embed_body,
        grid=(GRID,),
        in_specs=[
            pl.BlockSpec((ROWS, IN_DIM), lambda i: (i, 0)),
            pl.BlockSpec((ROWS, IN_DIM), lambda i: (i, 0)),
            pl.BlockSpec((IN_DIM, D), lambda i: (0, 0)),
            pl.BlockSpec((1, D), lambda i: (0, 0)),
        ],
        out_specs=[
            pl.BlockSpec((ROWS, D), lambda i: (i, 0)),
            pl.BlockSpec((ROWS, D), lambda i: (i, 0)),
        ],
        out_shape=[
            jax.ShapeDtypeStruct((NP, D), jnp.float32),
            jax.ShapeDtypeStruct((NP, D), jnp.float32),
        ],
    )(xp, zp, wcat, bcat)


def _update_body(h_ref, a_ref, zz_ref, ws_ref, wn_ref, out_ref):
    agg = a_ref[...] * INV_NBR
    pre = (jnp.dot(h_ref[...], ws_ref[...], preferred_element_type=jnp.float32)
           + jnp.dot(agg, wn_ref[...], preferred_element_type=jnp.float32))
    out_ref[...] = jax.nn.silu(pre) * zz_ref[...]


def _tc_update(h, agg, zz, ws_bd, wn_bd):
    return pl.pallas_call(
        _update_body,
        grid=(GRID,),
        in_specs=[
            pl.BlockSpec((ROWS, D), lambda i: (i, 0)),
            pl.BlockSpec((ROWS, D), lambda i: (i, 0)),
            pl.BlockSpec((ROWS, D), lambda i: (i, 0)),
            pl.BlockSpec((D, D), lambda i: (0, 0)),
            pl.BlockSpec((D, D), lambda i: (0, 0)),
        ],
        out_specs=pl.BlockSpec((ROWS, D), lambda i: (i, 0)),
        out_shape=jax.ShapeDtypeStruct((NP, D), jnp.float32),
    )(h, agg, zz, ws_bd, wn_bd)


def _head_body(h_ref, wo_ref, s_ref, g_ref, out_ref):
    o = jnp.dot(h_ref[...], wo_ref[...], preferred_element_type=jnp.float32)
    t = jnp.abs(jax.nn.silu(o))
    ss = jnp.dot(t * t, s_ref[...], preferred_element_type=jnp.float32)
    out_ref[...] = -(g_ref[...] * t) / (jnp.sqrt(ss) + 1e-8)


def _tc_head(h, wo_bd, smat, gvec):
    return pl.pallas_call(
        _head_body,
        grid=(GRID,),
        in_specs=[
            pl.BlockSpec((ROWS, D), lambda i: (i, 0)),
            pl.BlockSpec((D, DO), lambda i: (0, 0)),
            pl.BlockSpec((DO, DO), lambda i: (0, 0)),
            pl.BlockSpec((1, DO), lambda i: (0, 0)),
        ],
        out_specs=pl.BlockSpec((ROWS, DO), lambda i: (i, 0)),
        out_shape=jax.ShapeDtypeStruct((NP, DO), jnp.float32),
    )(h, wo_bd, smat, gvec)


def _sc_body(h_hbm, src_hbm, dst_hbm,
             agg_hbm, pairs_hbm, counts_hbm,
             src_v, dst_v, stg_v, my_v, pch_v, gsrc_v, gloc_v,
             rows_v, cnt_v, counts_v, acc_v, sem):
    c = lax.axis_index("c")
    s = lax.axis_index("s")
    w = c * NS + s
    lanes = lax.iota(jnp.int32, L)
    zvec = jnp.zeros((L,), jnp.float32)

    def _zero_acc():
        def _zrow(r, _):
            def _zcol(q, _):
                acc_v[r, pl.ds(q * L, L)] = zvec
                return 0
            return lax.fori_loop(0, D // L, _zcol, 0)
        lax.fori_loop(0, ACCR, _zrow, 0)
    _zero_acc()

    def block_body(bi, _):
        b = jnp.where(c == 0, bi, BLK_PER_SC + bi)
        valid_blk = b < NBLK
        base = b * NB

        @pl.when(valid_blk)
        def _():
            def chunk_body(k, total):
                eb = s * EPT + k * CHUNK
                pltpu.sync_copy(src_hbm.at[pl.ds(eb, CHUNK)], src_v)
                pltpu.sync_copy(dst_hbm.at[pl.ds(eb, CHUNK)], dst_v)

                def vreg_body(i, tot):
                    dvec = dst_v[pl.ds(i * L, L)]
                    svec = src_v[pl.ds(i * L, L)]
                    dloc = dvec - base
                    mask = (dloc >= 0) & (dloc < NB)
                    pref = plsc.cumsum(mask.astype(jnp.int32))
                    pos = (tot & (STG - 1)) + pref - 1
                    pack = (svec << 13) | dloc
                    plsc.store_scatter(stg_v, [pos], pack, mask=mask)
                    cnt = jnp.sum(mask.astype(jnp.int32))
                    new = tot + cnt

                    @pl.when((new >> 11) > (tot >> 11))
                    def _():
                        pltpu.sync_copy(
                            stg_v.at[pl.ds(0, STG)],
                            pairs_hbm.at[pl.ds(w * CAP + (tot >> 11) * STG, STG)])
                        stg_v[pl.ds(0, L)] = stg_v[pl.ds(STG, L)]
                    return new
                return lax.fori_loop(0, CHUNK // L, vreg_body, total)
            total = lax.fori_loop(0, NCHUNKS, chunk_body, 0)
            pltpu.sync_copy(stg_v.at[pl.ds(0, STG)],
                            pairs_hbm.at[pl.ds(w * CAP + (total >> 11) * STG, STG)])
            cnt_v[pl.ds(0, L)] = jnp.where(lanes == 0, total, 0)
            pltpu.sync_copy(cnt_v, counts_hbm.at[pl.ds(w * L, L)])
        plsc.subcore_barrier()

        @pl.when(valid_blk)
        def _():
            pltpu.sync_copy(counts_hbm.at[pl.ds(c * NS * L, NS * L)], counts_v)

            def consume(nbat, _):
                def one(j, _):
                    for t in range(K // L):
                        p = my_v[pl.ds(j * K + t * L, L)]
                        gsrc_v[pl.ds(t * L, L)] = p >> 13
                        gloc_v[pl.ds(t * L, L)] = p & 511
                    pltpu.async_copy(h_hbm.at[gsrc_v], rows_v, sem).wait()

                    def edge(i, _):
                        lvec = gloc_v[pl.ds((i >> 4) * L, L)]
                        dl = jnp.sum(jnp.where(lanes == (i & 15), lvec, 0))
                        for q in range(D // L):
                            plsc.addupdate(acc_v.at[dl, pl.ds(q * L, L)],
                                           rows_v[i, pl.ds(q * L, L)])
                        return 0
                    lax.fori_loop(0, K, edge, 0)
                    return 0
                lax.fori_loop(0, nbat, one, 0)
                return 0

            def seg_body(sg, mycnt):
                crow = counts_v[pl.ds(sg * L, L)]
                cnt_s = jnp.sum(jnp.where(lanes == 0, crow, 0))
                nch = (cnt_s + PCH - 1) >> 11

                def pch_body(ch, myc):
                    pltpu.sync_copy(
                        pairs_hbm.at[pl.ds((c * NS + sg) * CAP + ch * PCH,
                                           PCH)],
                        pch_v)
                    left = cnt_s - ch * PCH

                    def fil(i, mc):
                        p = pch_v[pl.ds(i * L, L)]
                        dloc = p & 8191
                        mine = ((dloc >> 8) == s) & (lanes < left - i * L)
                        pref = plsc.cumsum(mine.astype(jnp.int32))
                        pos = mc + pref - 1
                        plsc.store_scatter(my_v, [pos],
                                           (p & ~jnp.int32(8191)) | (dloc & 255),
                                           mask=mine)
                        return mc + jnp.sum(mine.astype(jnp.int32))
                    myc = lax.fori_loop(0, PCH // L, fil, myc)
                    nbat = myc >> 6
                    consume(nbat, 0)
                    rem = myc & 63

                    for t in range(K // L):
                        my_v[pl.ds(t * L, L)] = my_v[pl.ds(nbat * K + t * L, L)]
                    return rem
                return lax.fori_loop(0, nch, pch_body, mycnt)
            mycnt = lax.fori_loop(0, NS, seg_body, 0)

            @pl.when(mycnt > 0)
            def _():
                mix = s * 131 + bi * 29
                for t in range(K // L):
                    dummy = (((lanes + t * L + mix) & 16383) << 13) | \
                        (SUB + ((lanes + t) & 7))
                    pvec = my_v[pl.ds(t * L, L)]
                    keep = (t * L + lanes) < mycnt
                    my_v[pl.ds(t * L, L)] = jnp.where(keep, pvec, dummy)
                consume(1, 0)

            pltpu.sync_copy(acc_v.at[pl.ds(0, SUB)],
                            agg_hbm.at[pl.ds(base + s * SUB, SUB)])
            _zero_acc()
        plsc.subcore_barrier()
        return 0
    lax.fori_loop(0, BLK_PER_SC, block_body, 0)


def _sc_segment_sum(h, src, dst):
    mesh = plsc.VectorSubcoreMesh(core_axis_name="c", subcore_axis_name="s",
                                  num_cores=NC, num_subcores=NS)
    fn = pl.kernel(
        _sc_body,
        out_type=[
            jax.ShapeDtypeStruct((NP, D), jnp.float32),
            jax.ShapeDtypeStruct((NC * NS * CAP,), jnp.int32),
            jax.ShapeDtypeStruct((NC * NS * L,), jnp.int32),
        ],
        mesh=mesh,
        compiler_params=pltpu.CompilerParams(needs_layout_passes=False),
        scratch_types=[
            pltpu.VMEM((CHUNK,), jnp.int32),
            pltpu.VMEM((CHUNK,), jnp.int32),
            pltpu.VMEM((STG + L,), jnp.int32),
            pltpu.VMEM((PCH + K + L,), jnp.int32),
            pltpu.VMEM((PCH,), jnp.int32),
            pltpu.VMEM((K,), jnp.int32),
            pltpu.VMEM((K,), jnp.int32),
            pltpu.VMEM((K, D), jnp.float32),
            pltpu.VMEM((L,), jnp.int32),
            pltpu.VMEM((NS * L,), jnp.int32),
            pltpu.VMEM((ACCR, D), jnp.float32),
            pltpu.SemaphoreType.DMA,
        ],
    )
    return fn(h, src, dst)[0]


def kernel(x, z, edge_index, W_em, b_em, W_self, W_nbr, W_out, gamma):
    src = edge_index[0]
    dst = edge_index[1]

    eye = jnp.eye(M, dtype=jnp.float32)
    wcat = jnp.transpose(W_em, (1, 0, 2)).reshape(IN_DIM, D)
    bcat = b_em.reshape(1, D)
    ws_bd = [jnp.einsum("mij,mn->minj", W_self[:, l], eye).reshape(D, D)
             for l in range(2)]
    wn_bd = [jnp.einsum("mij,mn->minj", W_nbr[:, l], eye).reshape(D, D)
             for l in range(2)]
    wo_bd = jnp.einsum("mij,mn->minj", W_out, eye).reshape(D, DO)
    smat = jnp.einsum("mn,ij->minj", eye,
                      jnp.ones((ODIM, ODIM), jnp.float32)).reshape(DO, DO)
    gvec = jnp.repeat(gamma, ODIM).reshape(1, DO)

    pad = ((0, NP - N), (0, 0))
    xp = jnp.pad(x, pad)
    zp = jnp.pad(z, pad)

    h, zz = _tc_embed(xp, zp, wcat, bcat)
    for l in range(2):
        agg = _sc_segment_sum(h, src, dst)
        h = _tc_update(h, agg, zz, ws_bd[l], wn_bd[l])
    r = _tc_head(h, wo_bd, smat, gvec)

    r = r[:N].reshape(N, M, ODIM).transpose(0, 2, 1)
    r = r.reshape(N, ODIM, 2, ORB).transpose(0, 1, 3, 2)
    return r
